# baseline (speedup 1.0000x reference)
import jax
import jax.numpy as jnp
from jax import lax
from jax.experimental import pallas as pl
from jax.experimental.pallas import tpu as pltpu
from jax.experimental.pallas import tpu_sc as plsc

N = 10000
E = 320000
IN = 128
H = 256
OUT = 128
C = 4
G = 64

NP = 10240
BLK = 256
NB = NP // BLK
NSC = 2
NT = 16
RPT = NP // NT
K = 128

NCH = -(-E // NT // K)
EPT = NCH * K

NCH_D = -(-E // (NSC * NT) // K)
EPT_D = NCH_D * K


def _mesh():
    return plsc.VectorSubcoreMesh(core_axis_name="c", subcore_axis_name="s",
                                  num_cores=NSC, num_subcores=NT)


_SC_PARAMS = pltpu.CompilerParams(use_tc_tiling_on_sc=False)


def _deg_body(colD, zeros_hbm, ones_hbm, out_hbm, hist, cidx, ones_v):
    c = lax.axis_index("c")
    s = lax.axis_index("s")
    pltpu.sync_copy(zeros_hbm, hist.at[pl.ds(s * RPT, RPT)])
    pltpu.sync_copy(ones_hbm, ones_v)
    pltpu.sync_copy(colD.at[c, s], cidx)
    plsc.subcore_barrier()

    def body(g, carry):
        pltpu.sync_copy(ones_v, hist.at[cidx.at[g]], add=True)
        return carry

    lax.fori_loop(0, NCH_D, body, 0)
    plsc.subcore_barrier()
    pltpu.sync_copy(hist.at[pl.ds(s * RPT, RPT)],
                    out_hbm.at[pl.ds(c * NP + s * RPT, RPT)])


def _deg_call(colD, zeros16, ones16):
    return pl.kernel(
        _deg_body,
        out_type=jax.ShapeDtypeStruct((2 * NP, 16), jnp.float32),
        mesh=_mesh(),
        compiler_params=_SC_PARAMS,
        scratch_types=[
            pltpu.VMEM_SHARED((NP, 16), jnp.float32),
            pltpu.VMEM((NCH_D, K), jnp.int32),
            pltpu.VMEM((K, 16), jnp.float32),
        ],
    )(colD, zeros16, ones16)


def _agg_body(rc4, tab_a, tab_b, out_hbm,
              acc, rc, buf0, buf1,
              gsem0, gsem1, ssem0, ssem1, isem0, isem1, isem2, isem3):
    c = lax.axis_index("c")
    s = lax.axis_index("s")
    gsems = (gsem0, gsem1)
    ssems = (ssem0, ssem1)
    isems = (isem0, isem1, isem2, isem3)
    bufs = (buf0, buf1)

    def run(tab):
        pltpu.sync_copy(tab.at[pl.ds(s * RPT, RPT)],
                        acc.at[pl.ds(s * RPT, RPT)])
        plsc.subcore_barrier()
        pltpu.sync_copy(rc4.at[s, 0], rc.at[0])
        pltpu.async_copy(tab.at[rc.at[0, 0]], buf0, gsem0)
        pltpu.async_copy(rc4.at[s, 1], rc.at[1], isem1)

        def step(g, p, m):
            q = 1 - p
            m1 = (m + 1) % 4
            m2 = (m + 2) % 4
            m3 = (m + 3) % 4

            @pl.when(g + 1 < NCH)
            def _():
                pltpu.make_async_copy(rc4.at[s, g + 1], rc.at[m1],
                                      isems[m1]).wait()

            pltpu.make_async_copy(tab.at[rc.at[m, 0]], bufs[p],
                                  gsems[p]).wait()
            pltpu.async_copy(bufs[p], acc.at[rc.at[m, 1]], ssems[p],
                             add=True)

            @pl.when(g > 0)
            def _():
                pltpu.make_async_copy(bufs[q], acc.at[rc.at[m3, 1]],
                                      ssems[q]).wait()

            @pl.when(g + 1 < NCH)
            def _():
                pltpu.async_copy(tab.at[rc.at[m1, 0]], bufs[q], gsems[q])

            @pl.when(g + 2 < NCH)
            def _():
                pltpu.async_copy(rc4.at[s, g + 2], rc.at[m2], isems[m2])

        def body(rr, carry):
            g0 = 4 * rr
            for k in range(4):
                g = g0 + k

                @pl.when(g < NCH)
                def _():
                    step(g, k % 2, k)

            return carry

        lax.fori_loop(0, (NCH + 3) // 4, body, 0)
        gl = NCH - 1
        pltpu.make_async_copy(bufs[gl % 2], acc.at[rc.at[gl % 4, 1]],
                              ssems[gl % 2]).wait()
        plsc.subcore_barrier()
        pltpu.sync_copy(acc.at[pl.ds(s * RPT, RPT)],
                        out_hbm.at[pl.ds(c * NP + s * RPT, RPT)])

    @pl.when(c == 0)
    def _():
        run(tab_a)

    @pl.when(c == 1)
    def _():
        run(tab_b)


def _agg_call(Fh, rc4, tab_a, tab_b):
    return pl.kernel(
        _agg_body,
        out_type=jax.ShapeDtypeStruct((2 * NP, Fh), jnp.float32),
        mesh=_mesh(),
        compiler_params=_SC_PARAMS,
        scratch_types=[
            pltpu.VMEM_SHARED((NP, Fh), jnp.float32),
            pltpu.VMEM((4, 2, K), jnp.int32),
            pltpu.VMEM((K, Fh), jnp.float32),
            pltpu.VMEM((K, Fh), jnp.float32),
            pltpu.SemaphoreType.DMA,
            pltpu.SemaphoreType.DMA,
            pltpu.SemaphoreType.DMA,
            pltpu.SemaphoreType.DMA,
            pltpu.SemaphoreType.DMA,
            pltpu.SemaphoreType.DMA,
            pltpu.SemaphoreType.DMA,
            pltpu.SemaphoreType.DMA,
        ],
    )(rc4, tab_a, tab_b)


def _prescale_body(p0, p1, x, dis_b, y0a, y0b):
    deg = p0[:, 0:1] + p1[:, 0:1] + 1.0
    dis = lax.rsqrt(deg)
    dis_b[...] = jnp.broadcast_to(dis, (BLK, 128))
    y0a[...] = x[:, : IN // 2] * dis
    y0b[...] = x[:, IN // 2:] * dis


def _prescale_call(degp, xp):
    return pl.pallas_call(
        _prescale_body,
        grid=(NB,),
        in_specs=[
            pl.BlockSpec((BLK, 16), lambda b: (b, 0)),
            pl.BlockSpec((BLK, 16), lambda b: (b + NB, 0)),
            pl.BlockSpec((BLK, IN), lambda b: (b, 0)),
        ],
        out_specs=[
            pl.BlockSpec((BLK, 128), lambda b: (b, 0)),
            pl.BlockSpec((BLK, IN // 2), lambda b: (b, 0)),
            pl.BlockSpec((BLK, IN // 2), lambda b: (b, 0)),
        ],
        out_shape=[
            jax.ShapeDtypeStruct((NP, 128), jnp.float32),
            jax.ShapeDtypeStruct((NP, IN // 2), jnp.float32),
            jax.ShapeDtypeStruct((NP, IN // 2), jnp.float32),
        ],
    )(degp, degp, xp)


def _layer0_body(sa, sb, db, cb, W, Bp, ya, yb):
    dis = db[:, 0:1]
    a = sa[...] * dis
    b2 = sb[...] * dis
    cl = cb[:, 0:1]
    acc = jnp.zeros((BLK, H), jnp.float32)
    for j in range(C):
        m = (jnp.dot(a, W[j, : IN // 2, :], preferred_element_type=jnp.float32)
             + jnp.dot(b2, W[j, IN // 2:, :], preferred_element_type=jnp.float32)
             + Bp[j, :][None, :])
        acc = jnp.where(cl == j, m, acc)
    ya[...] = acc[:, : H // 2] * dis
    yb[...] = acc[:, H // 2:] * dis


def _layer0_call(s0, dis_b, cl_b, W0, B0p):
    return pl.pallas_call(
        _layer0_body,
        grid=(NB,),
        in_specs=[
            pl.BlockSpec((BLK, IN // 2), lambda b: (b, 0)),
            pl.BlockSpec((BLK, IN // 2), lambda b: (b + NB, 0)),
            pl.BlockSpec((BLK, 128), lambda b: (b, 0)),
            pl.BlockSpec((BLK, 128), lambda b: (b, 0)),
            pl.BlockSpec((C, IN, H), lambda b: (0, 0, 0)),
            pl.BlockSpec((8, H), lambda b: (0, 0)),
        ],
        out_specs=[
            pl.BlockSpec((BLK, H // 2), lambda b: (b, 0)),
            pl.BlockSpec((BLK, H // 2), lambda b: (b, 0)),
        ],
        out_shape=[
            jax.ShapeDtypeStruct((NP, H // 2), jnp.float32),
            jax.ShapeDtypeStruct((NP, H // 2), jnp.float32),
        ],
    )(s0, s0, dis_b, cl_b, W0, B0p)


def _layer1_body(sa, sb, db, cb, bt, W, Bp, Wm1, aux, Wm2, out, pooled):
    b = pl.program_id(0)
    dis = db[:, 0:1]
    agg = jnp.concatenate([sa[...] * dis, sb[...] * dis], axis=1)
    cl = cb[:, 0:1]
    acc = jnp.zeros((BLK, H), jnp.float32)
    for j in range(C):
        m = (jnp.dot(agg, W[j], preferred_element_type=jnp.float32)
             + Bp[j, :][None, :])
        acc = jnp.where(cl == j, m, acc)
    bids = bt[...].reshape(1, BLK)
    gi = lax.broadcasted_iota(jnp.int32, (G, BLK), 0)
    oh = (gi == bids).astype(jnp.float32)
    part = jnp.dot(oh, acc, preferred_element_type=jnp.float32)

    @pl.when(b == 0)
    def _():
        pooled[...] = part

    @pl.when(b > 0)
    def _():
        pooled[...] += part

    @pl.when(b == NB - 1)
    def _():
        p = pooled[...]
        hm = jnp.dot(p, Wm1[...], preferred_element_type=jnp.float32) + aux[0, :][None, :]
        mu = jnp.mean(hm, axis=0, keepdims=True)
        var = jnp.mean((hm - mu) * (hm - mu), axis=0, keepdims=True)
        hm = (hm - mu) * lax.rsqrt(var + 1e-5) * aux[1, :][None, :] + aux[2, :][None, :]
        hm = jnp.maximum(hm, 0.0)
        out[...] = jnp.dot(hm, Wm2[...], preferred_element_type=jnp.float32) + aux[3, :OUT][None, :]


def _layer1_call(s1, dis_b, cl_b, bt3, W1, B1p, Wm1, aux, Wm2):
    return pl.pallas_call(
        _layer1_body,
        grid=(NB,),
        in_specs=[
            pl.BlockSpec((BLK, H // 2), lambda b: (b, 0)),
            pl.BlockSpec((BLK, H // 2), lambda b: (b + NB, 0)),
            pl.BlockSpec((BLK, 128), lambda b: (b, 0)),
            pl.BlockSpec((BLK, 128), lambda b: (b, 0)),
            pl.BlockSpec((1, 1, BLK), lambda b: (b, 0, 0)),
            pl.BlockSpec((C, H, H), lambda b: (0, 0, 0)),
            pl.BlockSpec((8, H), lambda b: (0, 0)),
            pl.BlockSpec((H, H), lambda b: (0, 0)),
            pl.BlockSpec((8, H), lambda b: (0, 0)),
            pl.BlockSpec((H, OUT), lambda b: (0, 0)),
        ],
        out_specs=pl.BlockSpec((G, OUT), lambda b: (0, 0)),
        out_shape=jax.ShapeDtypeStruct((G, OUT), jnp.float32),
        scratch_shapes=[pltpu.VMEM((G, H), jnp.float32)],
    )(s1, s1, dis_b, cl_b, bt3, W1, B1p, Wm1, aux, Wm2)


def kernel(x_feat, cluster_labels, edge_index, batch, W0, B0, W1, B1,
           Wm1, bm1, gamma, beta, Wm2, bm2):
    row = edge_index[0].astype(jnp.int32)
    col = edge_index[1].astype(jnp.int32)
    row3 = jnp.pad(row.reshape(NT, E // NT), ((0, 0), (0, EPT - E // NT))
                   ).reshape(NT, NCH, K)
    col3 = jnp.pad(col.reshape(NT, E // NT), ((0, 0), (0, EPT - E // NT)),
                   constant_values=NP - 1).reshape(NT, NCH, K)
    rc4 = jnp.stack([row3, col3], axis=2)
    colD = jnp.pad(col, (0, NSC * NT * EPT_D - E),
                   constant_values=NP - 1).reshape(NSC, NT, NCH_D, K)

    xp = jnp.pad(x_feat, ((0, NP - N), (0, 0)))
    clp = jnp.pad(cluster_labels.astype(jnp.int32), (0, NP - N))
    cl_b = jnp.broadcast_to(clp[:, None], (NP, 128))
    btp = jnp.pad(batch.astype(jnp.int32), (0, NP - N), constant_values=G)
    bt3 = btp.reshape(NB, 1, BLK)
    B0p = jnp.pad(B0, ((0, 8 - C), (0, 0)))
    B1p = jnp.pad(B1, ((0, 8 - C), (0, 0)))
    aux = jnp.pad(
        jnp.stack([bm1, gamma, beta, jnp.pad(bm2, (0, H - OUT))], axis=0),
        ((0, 4), (0, 0)))
    zeros16 = jnp.zeros((RPT, 16), jnp.float32)
    ones16 = jnp.ones((K, 16), jnp.float32)

    degp = _deg_call(colD, zeros16, ones16)
    dis_b, y0a, y0b = _prescale_call(degp, xp)
    s0 = _agg_call(IN // 2, rc4, y0a, y0b)
    y1a, y1b = _layer0_call(s0, dis_b, cl_b, W0, B0p)
    s1 = _agg_call(H // 2, rc4, y1a, y1b)
    return _layer1_call(s1, dis_b, cl_b, bt3, W1, B1p, Wm1, aux, Wm2)


# speedup vs baseline: 1.0230x; 1.0230x over previous
import jax
import jax.numpy as jnp
from jax import lax
from jax.experimental import pallas as pl
from jax.experimental.pallas import tpu as pltpu
from jax.experimental.pallas import tpu_sc as plsc

N = 10000
E = 320000
IN = 128
H = 256
OUT = 128
C = 4
G = 64

NP = 10240
BLK = 256
NB = NP // BLK
NSC = 2
NT = 16
RPT = NP // NT
K = 128

NCH = -(-E // NT // K)
EPT = NCH * K

NCH_D = -(-E // (NSC * NT) // K)
EPT_D = NCH_D * K


def _mesh():
    return plsc.VectorSubcoreMesh(core_axis_name="c", subcore_axis_name="s",
                                  num_cores=NSC, num_subcores=NT)


_SC_PARAMS = pltpu.CompilerParams(use_tc_tiling_on_sc=False)


def _deg_body(colD, zeros_hbm, ones_hbm, out_hbm, hist, cidx, ones_v):
    c = lax.axis_index("c")
    s = lax.axis_index("s")
    pltpu.sync_copy(zeros_hbm, hist.at[pl.ds(s * RPT, RPT)])
    pltpu.sync_copy(ones_hbm, ones_v)
    pltpu.sync_copy(colD.at[c, s], cidx)
    plsc.subcore_barrier()

    def body(g, carry):
        pltpu.sync_copy(ones_v, hist.at[cidx.at[g]], add=True)
        return carry

    lax.fori_loop(0, NCH_D, body, 0)
    plsc.subcore_barrier()
    pltpu.sync_copy(hist.at[pl.ds(s * RPT, RPT)],
                    out_hbm.at[pl.ds(c * NP + s * RPT, RPT)])


def _deg_call(colD, zeros16, ones16):
    return pl.kernel(
        _deg_body,
        out_type=jax.ShapeDtypeStruct((2 * NP, 16), jnp.float32),
        mesh=_mesh(),
        compiler_params=_SC_PARAMS,
        scratch_types=[
            pltpu.VMEM_SHARED((NP, 16), jnp.float32),
            pltpu.VMEM((NCH_D, K), jnp.int32),
            pltpu.VMEM((K, 16), jnp.float32),
        ],
    )(colD, zeros16, ones16)


def _agg_body(rc4, tab_a, tab_b, out_hbm,
              acc, rc, buf0, buf1,
              gsem0, gsem1, isem0, isem1):
    c = lax.axis_index("c")
    s = lax.axis_index("s")
    gsems = (gsem0, gsem1)
    isems = (isem0, isem1)
    bufs = (buf0, buf1)

    def run(tab):
        pltpu.sync_copy(tab.at[pl.ds(s * RPT, RPT)],
                        acc.at[pl.ds(s * RPT, RPT)])
        plsc.subcore_barrier()
        pltpu.sync_copy(rc4.at[s, 0], rc.at[0])
        pltpu.async_copy(tab.at[rc.at[0, 0]], buf0, gsem0)
        pltpu.async_copy(rc4.at[s, 1], rc.at[1], isem1)

        def step(g, p):
            q = 1 - p
            bufp, bufq = (bufs[p], bufs[q])
            gsp, gsq = (gsems[p], gsems[q])
            isp, isq = (isems[p], isems[q])

            @pl.when(g + 1 < NCH)
            def _():
                pltpu.make_async_copy(rc4.at[s, g + 1], rc.at[q], isq).wait()
                pltpu.async_copy(tab.at[rc.at[q, 0]], bufq, gsq)

            pltpu.make_async_copy(tab.at[rc.at[p, 0]], bufp, gsp).wait()
            pltpu.sync_copy(bufp, acc.at[rc.at[p, 1]], add=True)

            @pl.when(g + 2 < NCH)
            def _():
                pltpu.async_copy(rc4.at[s, g + 2], rc.at[p], isp)

        def body(gg, carry):
            g0 = 2 * gg
            step(g0, 0)

            @pl.when(g0 + 1 < NCH)
            def _():
                step(g0 + 1, 1)

            return carry

        lax.fori_loop(0, (NCH + 1) // 2, body, 0)
        plsc.subcore_barrier()
        pltpu.sync_copy(acc.at[pl.ds(s * RPT, RPT)],
                        out_hbm.at[pl.ds(c * NP + s * RPT, RPT)])

    @pl.when(c == 0)
    def _():
        run(tab_a)

    @pl.when(c == 1)
    def _():
        run(tab_b)


def _agg_call(Fh, rc4, tab_a, tab_b):
    return pl.kernel(
        _agg_body,
        out_type=jax.ShapeDtypeStruct((2 * NP, Fh), jnp.float32),
        mesh=_mesh(),
        compiler_params=_SC_PARAMS,
        scratch_types=[
            pltpu.VMEM_SHARED((NP, Fh), jnp.float32),
            pltpu.VMEM((2, 2, K), jnp.int32),
            pltpu.VMEM((K, Fh), jnp.float32),
            pltpu.VMEM((K, Fh), jnp.float32),
            pltpu.SemaphoreType.DMA,
            pltpu.SemaphoreType.DMA,
            pltpu.SemaphoreType.DMA,
            pltpu.SemaphoreType.DMA,
        ],
    )(rc4, tab_a, tab_b)


def _prescale_body(p0, p1, x, dis_b, y0a, y0b):
    deg = p0[:, 0:1] + p1[:, 0:1] + 1.0
    dis = lax.rsqrt(deg)
    dis_b[...] = jnp.broadcast_to(dis, (BLK, 128))
    y0a[...] = x[:, : IN // 2] * dis
    y0b[...] = x[:, IN // 2:] * dis


def _prescale_call(degp, xp):
    return pl.pallas_call(
        _prescale_body,
        grid=(NB,),
        in_specs=[
            pl.BlockSpec((BLK, 16), lambda b: (b, 0)),
            pl.BlockSpec((BLK, 16), lambda b: (b + NB, 0)),
            pl.BlockSpec((BLK, IN), lambda b: (b, 0)),
        ],
        out_specs=[
            pl.BlockSpec((BLK, 128), lambda b: (b, 0)),
            pl.BlockSpec((BLK, IN // 2), lambda b: (b, 0)),
            pl.BlockSpec((BLK, IN // 2), lambda b: (b, 0)),
        ],
        out_shape=[
            jax.ShapeDtypeStruct((NP, 128), jnp.float32),
            jax.ShapeDtypeStruct((NP, IN // 2), jnp.float32),
            jax.ShapeDtypeStruct((NP, IN // 2), jnp.float32),
        ],
    )(degp, degp, xp)


def _layer0_body(sa, sb, db, cb, W, Bp, ya, yb):
    dis = db[:, 0:1]
    a = sa[...] * dis
    b2 = sb[...] * dis
    cl = cb[:, 0:1]
    acc = jnp.zeros((BLK, H), jnp.float32)
    for j in range(C):
        m = (jnp.dot(a, W[j, : IN // 2, :], preferred_element_type=jnp.float32)
             + jnp.dot(b2, W[j, IN // 2:, :], preferred_element_type=jnp.float32)
             + Bp[j, :][None, :])
        acc = jnp.where(cl == j, m, acc)
    ya[...] = acc[:, : H // 2] * dis
    yb[...] = acc[:, H // 2:] * dis


def _layer0_call(s0, dis_b, cl_b, W0, B0p):
    return pl.pallas_call(
        _layer0_body,
        grid=(NB,),
        in_specs=[
            pl.BlockSpec((BLK, IN // 2), lambda b: (b, 0)),
            pl.BlockSpec((BLK, IN // 2), lambda b: (b + NB, 0)),
            pl.BlockSpec((BLK, 128), lambda b: (b, 0)),
            pl.BlockSpec((BLK, 128), lambda b: (b, 0)),
            pl.BlockSpec((C, IN, H), lambda b: (0, 0, 0)),
            pl.BlockSpec((8, H), lambda b: (0, 0)),
        ],
        out_specs=[
            pl.BlockSpec((BLK, H // 2), lambda b: (b, 0)),
            pl.BlockSpec((BLK, H // 2), lambda b: (b, 0)),
        ],
        out_shape=[
            jax.ShapeDtypeStruct((NP, H // 2), jnp.float32),
            jax.ShapeDtypeStruct((NP, H // 2), jnp.float32),
        ],
    )(s0, s0, dis_b, cl_b, W0, B0p)


def _layer1_body(sa, sb, db, cb, bt, W, Bp, Wm1, aux, Wm2, out, pooled):
    b = pl.program_id(0)
    dis = db[:, 0:1]
    agg = jnp.concatenate([sa[...] * dis, sb[...] * dis], axis=1)
    cl = cb[:, 0:1]
    acc = jnp.zeros((BLK, H), jnp.float32)
    for j in range(C):
        m = (jnp.dot(agg, W[j], preferred_element_type=jnp.float32)
             + Bp[j, :][None, :])
        acc = jnp.where(cl == j, m, acc)
    bids = bt[...].reshape(1, BLK)
    gi = lax.broadcasted_iota(jnp.int32, (G, BLK), 0)
    oh = (gi == bids).astype(jnp.float32)
    part = jnp.dot(oh, acc, preferred_element_type=jnp.float32)

    @pl.when(b == 0)
    def _():
        pooled[...] = part

    @pl.when(b > 0)
    def _():
        pooled[...] += part

    @pl.when(b == NB - 1)
    def _():
        p = pooled[...]
        hm = jnp.dot(p, Wm1[...], preferred_element_type=jnp.float32) + aux[0, :][None, :]
        mu = jnp.mean(hm, axis=0, keepdims=True)
        var = jnp.mean((hm - mu) * (hm - mu), axis=0, keepdims=True)
        hm = (hm - mu) * lax.rsqrt(var + 1e-5) * aux[1, :][None, :] + aux[2, :][None, :]
        hm = jnp.maximum(hm, 0.0)
        out[...] = jnp.dot(hm, Wm2[...], preferred_element_type=jnp.float32) + aux[3, :OUT][None, :]


def _layer1_call(s1, dis_b, cl_b, bt3, W1, B1p, Wm1, aux, Wm2):
    return pl.pallas_call(
        _layer1_body,
        grid=(NB,),
        in_specs=[
            pl.BlockSpec((BLK, H // 2), lambda b: (b, 0)),
            pl.BlockSpec((BLK, H // 2), lambda b: (b + NB, 0)),
            pl.BlockSpec((BLK, 128), lambda b: (b, 0)),
            pl.BlockSpec((BLK, 128), lambda b: (b, 0)),
            pl.BlockSpec((1, 1, BLK), lambda b: (b, 0, 0)),
            pl.BlockSpec((C, H, H), lambda b: (0, 0, 0)),
            pl.BlockSpec((8, H), lambda b: (0, 0)),
            pl.BlockSpec((H, H), lambda b: (0, 0)),
            pl.BlockSpec((8, H), lambda b: (0, 0)),
            pl.BlockSpec((H, OUT), lambda b: (0, 0)),
        ],
        out_specs=pl.BlockSpec((G, OUT), lambda b: (0, 0)),
        out_shape=jax.ShapeDtypeStruct((G, OUT), jnp.float32),
        scratch_shapes=[pltpu.VMEM((G, H), jnp.float32)],
    )(s1, s1, dis_b, cl_b, bt3, W1, B1p, Wm1, aux, Wm2)


def kernel(x_feat, cluster_labels, edge_index, batch, W0, B0, W1, B1,
           Wm1, bm1, gamma, beta, Wm2, bm2):
    row = edge_index[0].astype(jnp.int32)
    col = edge_index[1].astype(jnp.int32)
    row3 = jnp.pad(row.reshape(NT, E // NT), ((0, 0), (0, EPT - E // NT))
                   ).reshape(NT, NCH, K)
    col3 = jnp.pad(col.reshape(NT, E // NT), ((0, 0), (0, EPT - E // NT)),
                   constant_values=NP - 1).reshape(NT, NCH, K)
    rc4 = jnp.stack([row3, col3], axis=2)
    colD = jnp.pad(col, (0, NSC * NT * EPT_D - E),
                   constant_values=NP - 1).reshape(NSC, NT, NCH_D, K)

    xp = jnp.pad(x_feat, ((0, NP - N), (0, 0)))
    clp = jnp.pad(cluster_labels.astype(jnp.int8), (0, NP - N))
    cl_b = jnp.broadcast_to(clp[:, None], (NP, 128))
    btp = jnp.pad(batch.astype(jnp.int32), (0, NP - N), constant_values=G)
    bt3 = btp.reshape(NB, 1, BLK)
    B0p = jnp.pad(B0, ((0, 8 - C), (0, 0)))
    B1p = jnp.pad(B1, ((0, 8 - C), (0, 0)))
    aux = jnp.pad(
        jnp.stack([bm1, gamma, beta, jnp.pad(bm2, (0, H - OUT))], axis=0),
        ((0, 4), (0, 0)))
    zeros16 = jnp.zeros((RPT, 16), jnp.float32)
    ones16 = jnp.ones((K, 16), jnp.float32)

    degp = _deg_call(colD, zeros16, ones16)
    dis_b, y0a, y0b = _prescale_call(degp, xp)
    s0 = _agg_call(IN // 2, rc4, y0a, y0b)
    y1a, y1b = _layer0_call(s0, dis_b, cl_b, W0, B0p)
    s1 = _agg_call(H // 2, rc4, y1a, y1b)
    return _layer1_call(s1, dis_b, cl_b, bt3, W1, B1p, Wm1, aux, Wm2)
